# baseline (speedup 1.0000x reference)
import functools
import jax
import jax.numpy as jnp
import numpy as np
from jax.experimental import pallas as pl
from jax.experimental.pallas import tpu as pltpu
from jax.experimental.pallas import tpu_sc as plsc


def _row_interleave(lx, ly, lz, B):
    lane = jax.lax.broadcasted_iota(jnp.int32, (1, 3 * B), 1)
    row = jnp.zeros((1, 3 * B), dtype=jnp.float32)
    for b in range(B):
        row = jnp.where(lane == 3 * b, jnp.broadcast_to(lx[b:b + 1, 0:1], (1, 3 * B)), row)
        row = jnp.where(lane == 3 * b + 1, jnp.broadcast_to(ly[b:b + 1, 0:1], (1, 3 * B)), row)
        row = jnp.where(lane == 3 * b + 2, jnp.broadcast_to(lz[b:b + 1, 0:1], (1, 3 * B)), row)
    return row


def _fps_body(px_ref, py_ref, pz_ref, out_ref, *, ns):
    px = px_ref[:, :]
    py = py_ref[:, :]
    pz = pz_ref[:, :]
    B, N = px.shape
    iota = jax.lax.broadcasted_iota(jnp.int32, (B, N), 1)

    lx = px[:, 0:1]
    ly = py[:, 0:1]
    lz = pz[:, 0:1]
    out_ref[0:1, :] = _row_interleave(lx, ly, lz, B)
    dists0 = jnp.full((B, N), jnp.inf, dtype=jnp.float32)

    def body(i, carry):
        dists, lx, ly, lz = carry
        d = (px - lx) ** 2 + (py - ly) ** 2 + (pz - lz) ** 2
        dists = jnp.minimum(dists, d)
        m = jnp.max(dists, axis=1, keepdims=True)
        sel = jnp.where(dists == m, iota, N)
        nxt = jnp.min(sel, axis=1, keepdims=True)
        onehot = iota == nxt
        lx = jnp.sum(jnp.where(onehot, px, 0.0), axis=1, keepdims=True)
        ly = jnp.sum(jnp.where(onehot, py, 0.0), axis=1, keepdims=True)
        lz = jnp.sum(jnp.where(onehot, pz, 0.0), axis=1, keepdims=True)
        out_ref[pl.ds(i, 1), :] = _row_interleave(lx, ly, lz, B)
        return dists, lx, ly, lz

    jax.lax.fori_loop(1, ns, body, (dists0, lx, ly, lz))


def _fps_pallas(pos, ns):
    B, N, _ = pos.shape
    px, py, pz = pos[:, :, 0], pos[:, :, 1], pos[:, :, 2]
    out = pl.pallas_call(
        functools.partial(_fps_body, ns=ns),
        out_shape=jax.ShapeDtypeStruct((ns, 3 * B), jnp.float32),
    )(px, py, pz)
    return out.reshape(ns, B, 3).transpose(1, 0, 2)


_EPS_D2 = 1e-4


def _sa_prep_body(posq_ref, pos_ref, x_ref, cm_ref, wa_ref, wb_ref, b1_ref,
                  sum_ref, f_ref, o_ref, *, r2, nqt):
    qt = pl.program_id(1)
    posq = posq_ref[0]
    pos = pos_ref[0]
    x = x_ref[0]
    q2 = jnp.sum(posq * posq, axis=1, keepdims=True)
    c2 = jnp.sum(pos * pos, axis=1, keepdims=True)
    ca = jnp.concatenate([-2.0 * pos, c2], axis=1)
    qa = jnp.concatenate([posq, jnp.ones_like(q2)], axis=1)
    d2m = jax.lax.dot_general(qa, ca, (((1,), (1,)), ((), ())),
                              preferred_element_type=jnp.float32) + q2
    mask = (d2m <= r2 + _EPS_D2).astype(jnp.float32)
    sum_ref[0] = jax.lax.dot_general(mask, cm_ref[...], (((1,), (0,)), ((), ())),
                                     preferred_element_type=jnp.float32)
    o_ref[0] = b1_ref[...] - jax.lax.dot_general(
        posq, wb_ref[...], (((1,), (0,)), ((), ())),
        preferred_element_type=jnp.float32)

    @pl.when(qt == 0)
    def _():
        f_ref[0] = (
            jax.lax.dot_general(x, wa_ref[...], (((1,), (0,)), ((), ())),
                                preferred_element_type=jnp.float32)
            + jax.lax.dot_general(pos, wb_ref[...], (((1,), (0,)), ((), ())),
                                  preferred_element_type=jnp.float32))


def _sa_prep(posq, pos, x, W1, b1, r, qt):
    B, NS, _ = posq.shape
    N = pos.shape[1]
    C = x.shape[2]
    H1 = W1.shape[1]
    NCH = N // 16
    nqt = NS // qt
    wa, wb = W1[:C], W1[C:]
    cm = (jnp.arange(N)[:, None] // 16 == jnp.arange(NCH)[None, :]).astype(jnp.float32)
    grid = (B, nqt)
    out = pl.pallas_call(
        functools.partial(_sa_prep_body, r2=r * r, nqt=nqt),
        grid=grid,
        in_specs=[
            pl.BlockSpec((1, qt, 3), lambda b, q: (b, q, 0)),
            pl.BlockSpec((1, N, 3), lambda b, q: (b, 0, 0)),
            pl.BlockSpec((1, N, C), lambda b, q: (b, 0, 0)),
            pl.BlockSpec((N, NCH), lambda b, q: (0, 0)),
            pl.BlockSpec((C, H1), lambda b, q: (0, 0)),
            pl.BlockSpec((3, H1), lambda b, q: (0, 0)),
            pl.BlockSpec((1, H1), lambda b, q: (0, 0)),
        ],
        out_specs=[
            pl.BlockSpec((1, qt, NCH), lambda b, q: (b, q, 0)),
            pl.BlockSpec((1, N, H1), lambda b, q: (b, 0, 0)),
            pl.BlockSpec((1, qt, H1), lambda b, q: (b, q, 0)),
        ],
        out_shape=[
            jax.ShapeDtypeStruct((B, NS, NCH), jnp.float32),
            jax.ShapeDtypeStruct((B, N, H1), jnp.float32),
            jax.ShapeDtypeStruct((B, NS, H1), jnp.float32),
        ],
    )(posq, pos, x, cm, wa, wb, b1.reshape(1, H1))
    return out


def _sa_sc_gather(qx, qy, qz, px, py, pz, summary, f_flat, *, B, N, NS, NCH, H1, r2):
    NW = 32
    RPW = (B * NS) // NW
    TPC = NW // B
    mesh = plsc.VectorSubcoreMesh(core_axis_name="c", subcore_axis_name="s")

    @functools.partial(
        pl.kernel, mesh=mesh,
        compiler_params=pltpu.CompilerParams(needs_layout_passes=False, use_tc_tiling_on_sc=False),
        out_type=[
            jax.ShapeDtypeStruct((B * NS * 64, H1), jnp.float32),
            jax.ShapeDtypeStruct((B * NS * 64,), jnp.float32),
        ],
        scratch_types=[
            pltpu.VMEM((RPW + 16,), jnp.float32),
            pltpu.VMEM((RPW + 16,), jnp.float32),
            pltpu.VMEM((RPW + 16,), jnp.float32),
            pltpu.VMEM((N,), jnp.float32),
            pltpu.VMEM((N,), jnp.float32),
            pltpu.VMEM((N,), jnp.float32),
            pltpu.VMEM((RPW * NCH,), jnp.float32),
            pltpu.VMEM((NCH + 16,), jnp.int32),
            pltpu.VMEM((96,), jnp.int32),
            pltpu.VMEM((RPW * 64 + 16,), jnp.float32),
            pltpu.VMEM((64,), jnp.int32),
            pltpu.VMEM((64,), jnp.int32),
            pltpu.VMEM((64,), jnp.int32),
            pltpu.VMEM((64,), jnp.int32),
            pltpu.VMEM((64, H1), jnp.float32),
            pltpu.VMEM((64, H1), jnp.float32),
            pltpu.VMEM((64, H1), jnp.float32),
            pltpu.VMEM((64, H1), jnp.float32),
            pltpu.SemaphoreType.DMA,
            pltpu.SemaphoreType.DMA,
            pltpu.SemaphoreType.DMA,
            pltpu.SemaphoreType.DMA,
            pltpu.SemaphoreType.DMA,
            pltpu.SemaphoreType.DMA,
            pltpu.SemaphoreType.DMA,
            pltpu.SemaphoreType.DMA,
        ],
    )
    def sc_kernel(qx_h, qy_h, qz_h, px_h, py_h, pz_h, sum_h, f_h, g_h, vm_h,
                  qxs, qys, qzs, pxs, pys, pzs, sums, hits, idxb, vmall,
                  i64a, i64b, i64c, i64d, rva, rvb, rvc, rvd,
                  sga, sgb, sgc, sgd, swa, swb, swc, swd):
        idx64s = [i64a, i64b, i64c, i64d]
        rows = [rva, rvb, rvc, rvd]
        semg = [sga, sgb, sgc, sgd]
        semw = [swa, swb, swc, swd]
        wid = jax.lax.axis_index("s") * 2 + jax.lax.axis_index("c")
        base = wid * RPW
        cl = wid // TPC
        pltpu.sync_copy(qx_h.at[pl.ds(base, RPW)], qxs.at[pl.ds(0, RPW)])
        pltpu.sync_copy(qy_h.at[pl.ds(base, RPW)], qys.at[pl.ds(0, RPW)])
        pltpu.sync_copy(qz_h.at[pl.ds(base, RPW)], qzs.at[pl.ds(0, RPW)])
        pltpu.sync_copy(px_h.at[pl.ds(cl * N, N)], pxs)
        pltpu.sync_copy(py_h.at[pl.ds(cl * N, N)], pys)
        pltpu.sync_copy(pz_h.at[pl.ds(cl * N, N)], pzs)
        pltpu.sync_copy(sum_h.at[pl.ds(base * NCH, RPW * NCH)], sums)
        iota = jax.lax.iota(jnp.int32, 16)
        ones16 = jnp.ones((16,), jnp.float32)
        dummy = jnp.full((16,), cl * N, jnp.int32)
        zeros16 = jnp.zeros((16,), jnp.float32)

        def compact(i, bx):
            for s in range(6):
                idxb[pl.ds(16 * s, 16)] = dummy
            for s in range(4):
                vmall[pl.ds(i * 64 + 16 * s, 16)] = zeros16
            nh = jnp.int32(0)
            for sc in range(NCH // 16):
                sv = sums[pl.ds(i * NCH + sc * 16, 16)]
                m = sv > 0.5
                plsc.store_compressed(hits.at[pl.ds(nh, 16)], iota + sc * 16, mask=m)
                nh = nh + plsc.all_reduce_population_count(m)[0]
            qxv = jnp.full((16,), qxs[pl.ds(i, 16)][0])
            qyv = jnp.full((16,), qys[pl.ds(i, 16)][0])
            qzv = jnp.full((16,), qzs[pl.ds(i, 16)][0])

            def cand_body(j2, cnt):
                off = hits[pl.ds(j2, 16)][0] * 16
                dx = pxs[pl.ds(off, 16)] - qxv
                dy = pys[pl.ds(off, 16)] - qyv
                dz = pzs[pl.ds(off, 16)] - qzv
                d2 = dx * dx + dy * dy + dz * dz
                mv = d2 <= r2
                npop = plsc.all_reduce_population_count(mv)[0]

                @pl.when(cnt < 64)
                def _():
                    plsc.store_compressed(idxb.at[pl.ds(cnt, 16)], iota + off + cl * N, mask=mv)
                    plsc.store_compressed(vmall.at[pl.ds(i * 64 + cnt, 16)], ones16, mask=mv)
                return cnt + npop

            jax.lax.fori_loop(0, nh, cand_body, jnp.int32(0))
            for s in range(4):
                idx64s[bx][pl.ds(16 * s, 16)] = idxb[pl.ds(16 * s, 16)]

        def wait_write(bx):
            pltpu.make_async_copy(rows[bx], g_h.at[pl.ds(0, 64)], semw[bx]).wait()

        def wait_gather(bx):
            pltpu.make_async_copy(f_h.at[idx64s[bx]], rows[bx], semg[bx]).wait()

        def start_write(bx, q):
            pltpu.async_copy(rows[bx], g_h.at[pl.ds(q * 64, 64)], semw[bx])

        def ring_body(i2, _):
            for bx in range(4):
                i = i2 * 4 + bx

                @pl.when(i2 > 0)
                def _():
                    wait_write(bx)
                compact(i, bx)
                pltpu.async_copy(f_h.at[idx64s[bx]], rows[bx], semg[bx])
                yx = (bx + 3) % 4
                if bx == 0:
                    @pl.when(i2 > 0)
                    def _():
                        wait_gather(yx)
                        start_write(yx, base + i - 1)
                else:
                    wait_gather(yx)
                    start_write(yx, base + i - 1)
            return 0

        jax.lax.fori_loop(0, RPW // 4, ring_body, 0)
        wait_gather(3)
        start_write(3, base + RPW - 1)
        for bx in range(4):
            wait_write(bx)
        pltpu.sync_copy(vmall.at[pl.ds(0, RPW * 64)], vm_h.at[pl.ds(base * 64, RPW * 64)])

    return sc_kernel(qx, qy, qz, px, py, pz, summary, f_flat)


def _sa_conv_body(g_ref, o_ref, vm_ref, w2_ref, b2_ref, w3_ref, b3_ref, out_ref):
    QT, H1 = o_ref.shape
    g = g_ref[...]
    o = jnp.broadcast_to(o_ref[...][:, None, :], (QT, 64, H1)).reshape(QT * 64, H1)
    h1 = jnp.maximum(g + o, 0.0)
    h2 = jnp.maximum(
        jax.lax.dot_general(h1, w2_ref[...], (((1,), (0,)), ((), ())),
                            preferred_element_type=jnp.float32) + b2_ref[...], 0.0)
    h3 = jax.lax.dot_general(h2, w3_ref[...], (((1,), (0,)), ((), ())),
                             preferred_element_type=jnp.float32) + b3_ref[...]
    H2 = h3.shape[1]
    vm = vm_ref[...]
    masked = jnp.where(vm[:, :, None] > 0.0, h3.reshape(QT, 64, H2),
                       -jnp.inf)
    out_ref[...] = jnp.max(masked, axis=1)


def _sa_conv(G, o_flat, vmask, W2, b2, W3, b3, qt):
    M, H1 = o_flat.shape
    H2 = W3.shape[1]
    grid = (M // qt,)
    return pl.pallas_call(
        _sa_conv_body,
        grid=grid,
        in_specs=[
            pl.BlockSpec((qt * 64, H1), lambda t: (t, 0)),
            pl.BlockSpec((qt, H1), lambda t: (t, 0)),
            pl.BlockSpec((qt, 64), lambda t: (t, 0)),
            pl.BlockSpec((H1, H1), lambda t: (0, 0)),
            pl.BlockSpec((1, H1), lambda t: (0, 0)),
            pl.BlockSpec((H1, H2), lambda t: (0, 0)),
            pl.BlockSpec((1, H2), lambda t: (0, 0)),
        ],
        out_specs=pl.BlockSpec((qt, H2), lambda t: (t, 0)),
        out_shape=jax.ShapeDtypeStruct((M, H2), jnp.float32),
    )(G, o_flat, vmask, W2, b2.reshape(1, H1), W3, b3.reshape(1, H2))


def _sa_pallas(x, pos, posq, mlp_params, r, prep_qt, conv_qt):
    B, N, C = x.shape
    NS = posq.shape[1]
    (W1, b1), (W2, b2), (W3, b3) = mlp_params
    H1 = W1.shape[1]
    summary, f, o = _sa_prep(posq, pos, x, W1, b1, r, prep_qt)
    G, vmask = _sa_sc_gather(
        posq[:, :, 0].reshape(-1), posq[:, :, 1].reshape(-1),
        posq[:, :, 2].reshape(-1),
        pos[:, :, 0].reshape(-1), pos[:, :, 1].reshape(-1),
        pos[:, :, 2].reshape(-1),
        summary.reshape(-1), f.reshape(B * N, H1),
        B=B, N=N, NS=NS, NCH=N // 16, H1=H1, r2=r * r)
    out = _sa_conv(G, o.reshape(B * NS, H1), vmask.reshape(B * NS, 64),
                   W2, b2, W3, b3, conv_qt)
    return out.reshape(B, NS, W3.shape[1])


def _td_body(x_ref, qc_ref, pt_ref, w_ref, b_ref, out_ref, *, k):
    x = x_ref[0]
    qc = qc_ref[0]
    pt = pt_ref[0]
    NS = qc.shape[0]
    N = x.shape[0]
    h = jnp.maximum(
        jax.lax.dot_general(x, w_ref[...], (((1,), (0,)), ((), ())),
                            preferred_element_type=jnp.float32) + b_ref[...], 0.0)
    d2 = ((qc[:, 0:1] - pt[0:1, :]) ** 2 + (qc[:, 1:2] - pt[1:2, :]) ** 2
          + (qc[:, 2:3] - pt[2:3, :]) ** 2)
    iota = jax.lax.broadcasted_iota(jnp.int32, (NS, N), 1)
    out = jnp.full((NS, h.shape[1]), -jnp.inf, dtype=jnp.float32)
    for _ in range(k):
        m = jnp.min(d2, axis=1, keepdims=True)
        j = jnp.min(jnp.where(d2 == m, iota, N), axis=1, keepdims=True)
        onehot = (iota == j).astype(jnp.float32)
        g = jax.lax.dot_general(onehot, h, (((1,), (0,)), ((), ())),
                                preferred_element_type=jnp.float32)
        out = jnp.maximum(out, g)
        d2 = jnp.where(iota == j, jnp.inf, d2)
    out_ref[0] = out


def _td_pallas(x, pos, posq, mlp_params, k=16):
    B, N, C = x.shape
    NS = posq.shape[1]
    (W, b) = mlp_params[0]
    H = W.shape[1]
    pos_t = pos.transpose(0, 2, 1)
    out = pl.pallas_call(
        functools.partial(_td_body, k=k),
        grid=(B,),
        in_specs=[
            pl.BlockSpec((1, N, C), lambda b_: (b_, 0, 0)),
            pl.BlockSpec((1, NS, 3), lambda b_: (b_, 0, 0)),
            pl.BlockSpec((1, 3, N), lambda b_: (b_, 0, 0)),
            pl.BlockSpec((C, H), lambda b_: (0, 0)),
            pl.BlockSpec((1, H), lambda b_: (0, 0)),
        ],
        out_specs=pl.BlockSpec((1, NS, H), lambda b_: (b_, 0, 0)),
        out_shape=jax.ShapeDtypeStruct((B, NS, H), jnp.float32),
    )(x, posq, pos_t, W, b.reshape(1, H))
    return out


def _cheb_body(x_ref, pc_ref, pt_ref, w0_ref, w1_ref, w2_ref, b_ref, eye_ref,
               out_ref, *, k):
    x = x_ref[0]
    pc = pc_ref[0]
    pt = pt_ref[0]
    N = x.shape[0]
    d2 = ((pc[:, 0:1] - pt[0:1, :]) ** 2 + (pc[:, 1:2] - pt[1:2, :]) ** 2
          + (pc[:, 2:3] - pt[2:3, :]) ** 2)
    iota = jax.lax.broadcasted_iota(jnp.int32, (N, N), 1)
    adj = jnp.zeros((N, N), dtype=jnp.float32)
    for _ in range(k):
        m = jnp.min(d2, axis=1, keepdims=True)
        j = jnp.min(jnp.where(d2 == m, iota, N), axis=1, keepdims=True)
        adj = adj + (iota == j).astype(jnp.float32)
        d2 = jnp.where(iota == j, jnp.inf, d2)
    eye = eye_ref[...]
    adj = adj * (1.0 - eye)
    deg = jnp.sum(adj, axis=1, keepdims=True)
    dinv = jnp.where(deg > 0.0, jax.lax.rsqrt(jnp.maximum(deg, 1e-12)), 0.0)
    dinv_row = jax.lax.dot_general(dinv, eye, (((0,), (0,)), ((), ())),
                                   preferred_element_type=jnp.float32)
    a = -dinv * dinv_row * adj

    def lhat(y):
        return jax.lax.dot_general(a, y, (((0,), (0,)), ((), ())),
                                   preferred_element_type=jnp.float32)

    tx1 = lhat(x)
    tx2 = 2.0 * lhat(tx1) - x

    def mm(u, wref):
        return jax.lax.dot_general(u, wref[...], (((1,), (0,)), ((), ())),
                                   preferred_element_type=jnp.float32)

    out_ref[0] = mm(x, w0_ref) + mm(tx1, w1_ref) + mm(tx2, w2_ref) + b_ref[...]


def _cheb_pallas(x, pos, params, k=16):
    B, N, C = x.shape
    Ws = params["Ws"]
    H = Ws[0].shape[1]
    pos_t = pos.transpose(0, 2, 1)
    eye = jnp.eye(N, dtype=jnp.float32)
    out = pl.pallas_call(
        functools.partial(_cheb_body, k=k),
        grid=(B,),
        in_specs=[
            pl.BlockSpec((1, N, C), lambda b_: (b_, 0, 0)),
            pl.BlockSpec((1, N, 3), lambda b_: (b_, 0, 0)),
            pl.BlockSpec((1, 3, N), lambda b_: (b_, 0, 0)),
            pl.BlockSpec((C, H), lambda b_: (0, 0)),
            pl.BlockSpec((C, H), lambda b_: (0, 0)),
            pl.BlockSpec((C, H), lambda b_: (0, 0)),
            pl.BlockSpec((1, H), lambda b_: (0, 0)),
            pl.BlockSpec((N, N), lambda b_: (0, 0)),
        ],
        out_specs=pl.BlockSpec((1, N, H), lambda b_: (b_, 0, 0)),
        out_shape=jax.ShapeDtypeStruct((B, N, H), jnp.float32),
    )(x, pos, pos_t, Ws[0], Ws[1], Ws[2], params["b"].reshape(1, H), eye)
    return out


def _head_body(x_ref, *refs):
    wrefs = refs[:-1]
    out_ref = refs[-1]
    h = x_ref[...]
    nw = len(wrefs) // 2

    def mm(u, wref):
        return jax.lax.dot_general(u, wref[...], (((1,), (0,)), ((), ())),
                                   preferred_element_type=jnp.float32)

    for li in range(3):
        h = mm(h, wrefs[2 * li]) + wrefs[2 * li + 1][...]
        if li < 2:
            h = jnp.maximum(h, 0.0)
    M = h.shape[0]
    B = M // 32
    g = jnp.mean(h.reshape(B, 32, h.shape[1]), axis=1)
    for li in range(3, nw):
        g = mm(g, wrefs[2 * li]) + wrefs[2 * li + 1][...]
        if li < nw - 1:
            g = jnp.maximum(g, 0.0)
    out_ref[...] = g


def _head_pallas(xp, sa3_params, head_params):
    B = xp.shape[0]
    X = xp.reshape(B * 32, xp.shape[2])
    wlist = []
    specs = [pl.BlockSpec(X.shape, lambda: (0, 0))]
    for (W, b) in list(sa3_params) + list(head_params):
        wlist += [W, b.reshape(1, W.shape[1])]
        specs += [pl.BlockSpec(W.shape, lambda: (0, 0)),
                  pl.BlockSpec((1, W.shape[1]), lambda: (0, 0))]
    H = wlist[-1].shape[1]
    return pl.pallas_call(
        _head_body,
        in_specs=specs,
        out_specs=pl.BlockSpec((B, H), lambda: (0, 0)),
        out_shape=jax.ShapeDtypeStruct((B, H), jnp.float32),
    )(X, *wlist)


def _mlp(params, x, plain_last=True):
    n = len(params)
    for i, (W, b) in enumerate(params):
        x = x @ W + b
        if (i < n - 1) or (not plain_last):
            x = jax.nn.relu(x)
    return x


def _fps(pos, n_sample):
    pos = jax.lax.stop_gradient(pos)
    N = pos.shape[0]
    idxs = jnp.zeros((n_sample,), dtype=jnp.int32)
    dists = jnp.full((N,), jnp.inf, dtype=jnp.float32)

    def body(i, carry):
        idxs, dists = carry
        last = pos[idxs[i - 1]]
        d = jnp.sum((pos - last) ** 2, axis=-1)
        dists = jnp.minimum(dists, d)
        nxt = jnp.argmax(dists).astype(jnp.int32)
        return idxs.at[i].set(nxt), dists

    idxs, _ = jax.lax.fori_loop(1, n_sample, body, (idxs, dists))
    return idxs


def _pairwise_sq(a, b):
    return jnp.sum((a[:, None, :] - b[None, :, :]) ** 2, axis=-1)


def _sa(x, pos, posq, mlp_params, r, max_nbr=64):
    d2 = _pairwise_sq(posq, pos)
    neg = jnp.where(d2 <= r * r, -d2, -jnp.inf)
    vals, nbr = jax.lax.top_k(neg, max_nbr)
    valid = vals > -jnp.inf
    msg = jnp.concatenate([x[nbr], pos[nbr] - posq[:, None, :]], axis=-1)
    h = _mlp(mlp_params, msg, plain_last=True)
    h = jnp.where(valid[:, :, None], h, -jnp.inf)
    out = jnp.max(h, axis=1)
    return out, posq


def _td(x, pos, posq, mlp_params, k=16):
    h = _mlp(mlp_params, x, plain_last=False)
    d2 = _pairwise_sq(posq, pos)
    _, nbr = jax.lax.top_k(-d2, k)
    out = jnp.max(h[nbr], axis=1)
    return out, posq


def _cheb(x, pos, params, k=16):
    n = x.shape[0]
    d2 = _pairwise_sq(pos, pos)
    _, nbr = jax.lax.top_k(-d2, k)
    src = jnp.repeat(jnp.arange(n), k)
    dst = nbr.reshape(-1)
    mask = (src != dst).astype(jnp.float32)
    deg = jax.ops.segment_sum(mask, src, num_segments=n)
    dinv = jnp.where(deg > 0, jax.lax.rsqrt(jnp.maximum(deg, 1e-12)), 0.0)
    w = -dinv[src] * dinv[dst] * mask

    def lhat(y):
        return jax.ops.segment_sum(w[:, None] * y[src], dst, num_segments=n)

    tx0 = x
    tx1 = lhat(tx0)
    tx2 = 2.0 * lhat(tx1) - tx0
    Ws = params["Ws"]
    return tx0 @ Ws[0] + tx1 @ Ws[1] + tx2 @ Ws[2] + params["b"]


def kernel(data, params):
    pos = data
    x = data
    q1 = _fps_pallas(pos, 2048)
    x = _sa_pallas(x, pos, q1, params["sa1"], 0.2, prep_qt=256, conv_qt=256)
    pos = q1
    q2 = _fps_pallas(pos, 512)
    x = _td_pallas(x, pos, q2, params["td1"])
    pos = q2
    x = _cheb_pallas(x, pos, params["cheb1"])
    q3 = _fps_pallas(pos, 128)
    x = _sa_pallas(x, pos, q3, params["sa2"], 0.4, prep_qt=128, conv_qt=128)
    pos = q3
    q4 = _fps_pallas(pos, 32)
    x = _td_pallas(x, pos, q4, params["td2"])
    pos = q4
    x = _cheb_pallas(x, pos, params["cheb2"])
    y = _head_pallas(jnp.concatenate([x, pos], axis=-1), params["sa3"], params["head"])
    return y


# speedup vs baseline: 1.7946x; 1.7946x over previous
import functools
import jax
import jax.numpy as jnp
import numpy as np
from jax.experimental import pallas as pl
from jax.experimental.pallas import tpu as pltpu
from jax.experimental.pallas import tpu_sc as plsc


def _row_interleave(lx, ly, lz, B):
    lane = jax.lax.broadcasted_iota(jnp.int32, (1, 3 * B), 1)
    row = jnp.zeros((1, 3 * B), dtype=jnp.float32)
    for b in range(B):
        row = jnp.where(lane == 3 * b, jnp.broadcast_to(lx[b:b + 1, 0:1], (1, 3 * B)), row)
        row = jnp.where(lane == 3 * b + 1, jnp.broadcast_to(ly[b:b + 1, 0:1], (1, 3 * B)), row)
        row = jnp.where(lane == 3 * b + 2, jnp.broadcast_to(lz[b:b + 1, 0:1], (1, 3 * B)), row)
    return row


def _fps_body(px_ref, py_ref, pz_ref, out_ref, *, ns):
    px = px_ref[:, :]
    py = py_ref[:, :]
    pz = pz_ref[:, :]
    B, N = px.shape
    iota = jax.lax.broadcasted_iota(jnp.int32, (B, N), 1)

    lx = px[:, 0:1]
    ly = py[:, 0:1]
    lz = pz[:, 0:1]
    out_ref[0:1, :] = _row_interleave(lx, ly, lz, B)
    dists0 = jnp.full((B, N), jnp.inf, dtype=jnp.float32)

    def body(i, carry):
        dists, lx, ly, lz = carry
        d = (px - lx) ** 2 + (py - ly) ** 2 + (pz - lz) ** 2
        dists = jnp.minimum(dists, d)
        m = jnp.max(dists, axis=1, keepdims=True)
        sel = jnp.where(dists == m, iota, N)
        nxt = jnp.min(sel, axis=1, keepdims=True)
        onehot = iota == nxt
        lx = jnp.sum(jnp.where(onehot, px, 0.0), axis=1, keepdims=True)
        ly = jnp.sum(jnp.where(onehot, py, 0.0), axis=1, keepdims=True)
        lz = jnp.sum(jnp.where(onehot, pz, 0.0), axis=1, keepdims=True)
        out_ref[pl.ds(i, 1), :] = _row_interleave(lx, ly, lz, B)
        return dists, lx, ly, lz

    jax.lax.fori_loop(1, ns, body, (dists0, lx, ly, lz))


def _fps_pallas(pos, ns):
    B, N, _ = pos.shape
    px, py, pz = pos[:, :, 0], pos[:, :, 1], pos[:, :, 2]
    out = pl.pallas_call(
        functools.partial(_fps_body, ns=ns),
        out_shape=jax.ShapeDtypeStruct((ns, 3 * B), jnp.float32),
    )(px, py, pz)
    return out.reshape(ns, B, 3).transpose(1, 0, 2)


_EPS_D2 = 1e-4


def _sa_prep_body(posq_ref, pos_ref, x_ref, cm_ref, wa_ref, wb_ref, b1_ref,
                  sum_ref, f_ref, o_ref, *, r2, nqt):
    qt = pl.program_id(1)
    posq = posq_ref[0]
    pos = pos_ref[0]
    x = x_ref[0]
    q2 = jnp.sum(posq * posq, axis=1, keepdims=True)
    c2 = jnp.sum(pos * pos, axis=1, keepdims=True)
    ca = jnp.concatenate([-2.0 * pos, c2], axis=1)
    qa = jnp.concatenate([posq, jnp.ones_like(q2)], axis=1)
    d2m = jax.lax.dot_general(qa, ca, (((1,), (1,)), ((), ())),
                              preferred_element_type=jnp.float32) + q2
    mask = (d2m <= r2 + _EPS_D2).astype(jnp.float32)
    sum_ref[0] = jax.lax.dot_general(mask, cm_ref[...], (((1,), (0,)), ((), ())),
                                     preferred_element_type=jnp.float32)
    o_ref[0] = b1_ref[...] - jax.lax.dot_general(
        posq, wb_ref[...], (((1,), (0,)), ((), ())),
        preferred_element_type=jnp.float32)

    @pl.when(qt == 0)
    def _():
        f_ref[0] = (
            jax.lax.dot_general(x, wa_ref[...], (((1,), (0,)), ((), ())),
                                preferred_element_type=jnp.float32)
            + jax.lax.dot_general(pos, wb_ref[...], (((1,), (0,)), ((), ())),
                                  preferred_element_type=jnp.float32))


def _sa_prep(posq, pos, x, W1, b1, r, qt):
    B, NS, _ = posq.shape
    N = pos.shape[1]
    C = x.shape[2]
    H1 = W1.shape[1]
    NCH = N // 16
    nqt = NS // qt
    wa, wb = W1[:C], W1[C:]
    cm = (jnp.arange(N)[:, None] // 16 == jnp.arange(NCH)[None, :]).astype(jnp.float32)
    grid = (B, nqt)
    out = pl.pallas_call(
        functools.partial(_sa_prep_body, r2=r * r, nqt=nqt),
        grid=grid,
        in_specs=[
            pl.BlockSpec((1, qt, 3), lambda b, q: (b, q, 0)),
            pl.BlockSpec((1, N, 3), lambda b, q: (b, 0, 0)),
            pl.BlockSpec((1, N, C), lambda b, q: (b, 0, 0)),
            pl.BlockSpec((N, NCH), lambda b, q: (0, 0)),
            pl.BlockSpec((C, H1), lambda b, q: (0, 0)),
            pl.BlockSpec((3, H1), lambda b, q: (0, 0)),
            pl.BlockSpec((1, H1), lambda b, q: (0, 0)),
        ],
        out_specs=[
            pl.BlockSpec((1, qt, NCH), lambda b, q: (b, q, 0)),
            pl.BlockSpec((1, N, H1), lambda b, q: (b, 0, 0)),
            pl.BlockSpec((1, qt, H1), lambda b, q: (b, q, 0)),
        ],
        out_shape=[
            jax.ShapeDtypeStruct((B, NS, NCH), jnp.float32),
            jax.ShapeDtypeStruct((B, N, H1), jnp.float32),
            jax.ShapeDtypeStruct((B, NS, H1), jnp.float32),
        ],
    )(posq, pos, x, cm, wa, wb, b1.reshape(1, H1))
    return out


def _sa_sc_gather(qx, qy, qz, px, py, pz, summary, f_flat, *, B, N, NS, NCH, H1, r2):
    NW = 32
    RPW = (B * NS) // NW
    TPC = NW // B
    mesh = plsc.VectorSubcoreMesh(core_axis_name="c", subcore_axis_name="s")

    @functools.partial(
        pl.kernel, mesh=mesh,
        compiler_params=pltpu.CompilerParams(needs_layout_passes=False, use_tc_tiling_on_sc=False),
        out_type=[
            jax.ShapeDtypeStruct((B * NS * 64, H1), jnp.float32),
            jax.ShapeDtypeStruct((B * NS * 64,), jnp.float32),
        ],
        scratch_types=[
            pltpu.VMEM((RPW + 16,), jnp.float32),
            pltpu.VMEM((RPW + 16,), jnp.float32),
            pltpu.VMEM((RPW + 16,), jnp.float32),
            pltpu.VMEM((N,), jnp.float32),
            pltpu.VMEM((N,), jnp.float32),
            pltpu.VMEM((N,), jnp.float32),
            pltpu.VMEM((RPW * NCH,), jnp.float32),
            pltpu.VMEM((NCH + 16,), jnp.int32),
            pltpu.VMEM((96,), jnp.int32),
            pltpu.VMEM((RPW * 64 + 16,), jnp.float32),
            pltpu.VMEM((64,), jnp.int32),
            pltpu.VMEM((64,), jnp.int32),
            pltpu.VMEM((64,), jnp.int32),
            pltpu.VMEM((64,), jnp.int32),
            pltpu.VMEM((64, H1), jnp.float32),
            pltpu.VMEM((64, H1), jnp.float32),
            pltpu.VMEM((64, H1), jnp.float32),
            pltpu.VMEM((64, H1), jnp.float32),
            pltpu.SemaphoreType.DMA,
            pltpu.SemaphoreType.DMA,
            pltpu.SemaphoreType.DMA,
            pltpu.SemaphoreType.DMA,
            pltpu.SemaphoreType.DMA,
            pltpu.SemaphoreType.DMA,
            pltpu.SemaphoreType.DMA,
            pltpu.SemaphoreType.DMA,
        ],
    )
    def sc_kernel(qx_h, qy_h, qz_h, px_h, py_h, pz_h, sum_h, f_h, g_h, vm_h,
                  qxs, qys, qzs, pxs, pys, pzs, sums, hits, idxb, vmall,
                  i64a, i64b, i64c, i64d, rva, rvb, rvc, rvd,
                  sga, sgb, sgc, sgd, swa, swb, swc, swd):
        idx64s = [i64a, i64b, i64c, i64d]
        rows = [rva, rvb, rvc, rvd]
        semg = [sga, sgb, sgc, sgd]
        semw = [swa, swb, swc, swd]
        wid = jax.lax.axis_index("s") * 2 + jax.lax.axis_index("c")
        base = wid * RPW
        cl = wid // TPC
        pltpu.sync_copy(qx_h.at[pl.ds(base, RPW)], qxs.at[pl.ds(0, RPW)])
        pltpu.sync_copy(qy_h.at[pl.ds(base, RPW)], qys.at[pl.ds(0, RPW)])
        pltpu.sync_copy(qz_h.at[pl.ds(base, RPW)], qzs.at[pl.ds(0, RPW)])
        pltpu.sync_copy(px_h.at[pl.ds(cl * N, N)], pxs)
        pltpu.sync_copy(py_h.at[pl.ds(cl * N, N)], pys)
        pltpu.sync_copy(pz_h.at[pl.ds(cl * N, N)], pzs)
        pltpu.sync_copy(sum_h.at[pl.ds(base * NCH, RPW * NCH)], sums)
        iota = jax.lax.iota(jnp.int32, 16)
        ones16 = jnp.ones((16,), jnp.float32)
        dummy = jnp.full((16,), cl * N, jnp.int32)
        zeros16 = jnp.zeros((16,), jnp.float32)

        def compact(i, bx):
            for s in range(6):
                idxb[pl.ds(16 * s, 16)] = dummy
            for s in range(4):
                vmall[pl.ds(i * 64 + 16 * s, 16)] = zeros16
            nh = jnp.int32(0)
            for sc in range(NCH // 16):
                sv = sums[pl.ds(i * NCH + sc * 16, 16)]
                m = sv > 0.5
                plsc.store_compressed(hits.at[pl.ds(nh, 16)], iota + sc * 16, mask=m)
                nh = nh + plsc.all_reduce_population_count(m)[0]
            qxv = jnp.full((16,), qxs[pl.ds(i, 16)][0])
            qyv = jnp.full((16,), qys[pl.ds(i, 16)][0])
            qzv = jnp.full((16,), qzs[pl.ds(i, 16)][0])

            def cand_body(j2, cnt):
                off = hits[pl.ds(j2, 16)][0] * 16
                dx = pxs[pl.ds(off, 16)] - qxv
                dy = pys[pl.ds(off, 16)] - qyv
                dz = pzs[pl.ds(off, 16)] - qzv
                d2 = dx * dx + dy * dy + dz * dz
                mv = d2 <= r2
                npop = plsc.all_reduce_population_count(mv)[0]

                @pl.when(cnt < 64)
                def _():
                    plsc.store_compressed(idxb.at[pl.ds(cnt, 16)], iota + off + cl * N, mask=mv)
                    plsc.store_compressed(vmall.at[pl.ds(i * 64 + cnt, 16)], ones16, mask=mv)
                return cnt + npop

            jax.lax.fori_loop(0, nh, cand_body, jnp.int32(0))
            for s in range(4):
                idx64s[bx][pl.ds(16 * s, 16)] = idxb[pl.ds(16 * s, 16)]

        def wait_write(bx):
            pltpu.make_async_copy(rows[bx], g_h.at[pl.ds(0, 64)], semw[bx]).wait()

        def wait_gather(bx):
            pltpu.make_async_copy(f_h.at[idx64s[bx]], rows[bx], semg[bx]).wait()

        def start_write(bx, q):
            pltpu.async_copy(rows[bx], g_h.at[pl.ds(q * 64, 64)], semw[bx])

        def ring_body(i2, _):
            for bx in range(4):
                i = i2 * 4 + bx

                @pl.when(i2 > 0)
                def _():
                    wait_write(bx)
                compact(i, bx)
                pltpu.async_copy(f_h.at[idx64s[bx]], rows[bx], semg[bx])
                yx = (bx + 3) % 4
                if bx == 0:
                    @pl.when(i2 > 0)
                    def _():
                        wait_gather(yx)
                        start_write(yx, base + i - 1)
                else:
                    wait_gather(yx)
                    start_write(yx, base + i - 1)
            return 0

        jax.lax.fori_loop(0, RPW // 4, ring_body, 0)
        wait_gather(3)
        start_write(3, base + RPW - 1)
        for bx in range(4):
            wait_write(bx)
        pltpu.sync_copy(vmall.at[pl.ds(0, RPW * 64)], vm_h.at[pl.ds(base * 64, RPW * 64)])

    return sc_kernel(qx, qy, qz, px, py, pz, summary, f_flat)


def _sa_conv_body(g_ref, o_ref, vm_ref, w2_ref, b2_ref, w3_ref, b3_ref, out_ref,
                  *, wab_refs=None):
    QT, H1 = o_ref.shape
    g = g_ref[...]
    o = jnp.broadcast_to(o_ref[...][:, None, :], (QT, 64, H1)).reshape(QT * 64, H1)
    if wab_refs is not None:
        wa_ref, wb_ref = wab_refs
        wsum = wa_ref[...] + wb_ref[...]
        g = jax.lax.dot_general(g[:, 0:3], wsum, (((1,), (0,)), ((), ())),
                                preferred_element_type=jnp.float32)
    h1 = jnp.maximum(g + o, 0.0)
    h2 = jnp.maximum(
        jax.lax.dot_general(h1, w2_ref[...], (((1,), (0,)), ((), ())),
                            preferred_element_type=jnp.float32) + b2_ref[...], 0.0)
    h3 = jax.lax.dot_general(h2, w3_ref[...], (((1,), (0,)), ((), ())),
                             preferred_element_type=jnp.float32) + b3_ref[...]
    H2 = h3.shape[1]
    vm = vm_ref[...]
    masked = jnp.where(vm[:, :, None] > 0.0, h3.reshape(QT, 64, H2),
                       -jnp.inf)
    out_ref[...] = jnp.max(masked, axis=1)


def _sa_conv(G, o_flat, vmask, W2, b2, W3, b3, qt, wab=None):
    M, H1 = o_flat.shape
    H2 = W3.shape[1]
    GW = G.shape[1]
    grid = (M // qt,)
    specs = [
        pl.BlockSpec((qt * 64, GW), lambda t: (t, 0)),
        pl.BlockSpec((qt, H1), lambda t: (t, 0)),
        pl.BlockSpec((qt, 64), lambda t: (t, 0)),
        pl.BlockSpec((H1, H1), lambda t: (0, 0)),
        pl.BlockSpec((1, H1), lambda t: (0, 0)),
        pl.BlockSpec((H1, H2), lambda t: (0, 0)),
        pl.BlockSpec((1, H2), lambda t: (0, 0)),
    ]
    args = [G, o_flat, vmask, W2, b2.reshape(1, H1), W3, b3.reshape(1, H2)]
    body = _sa_conv_body
    if wab is not None:
        specs += [pl.BlockSpec((3, H1), lambda t: (0, 0)),
                  pl.BlockSpec((3, H1), lambda t: (0, 0))]
        args += [wab[0], wab[1]]

        def body(g_ref, o_ref, vm_ref, w2_ref, b2_ref, w3_ref, b3_ref,
                 wa_ref, wb_ref, out_ref):
            return _sa_conv_body(g_ref, o_ref, vm_ref, w2_ref, b2_ref,
                                 w3_ref, b3_ref, out_ref,
                                 wab_refs=(wa_ref, wb_ref))
    return pl.pallas_call(
        body,
        grid=grid,
        in_specs=specs,
        out_specs=pl.BlockSpec((qt, H2), lambda t: (t, 0)),
        out_shape=jax.ShapeDtypeStruct((M, H2), jnp.float32),
    )(*args)


def _sa_pallas(x, pos, posq, mlp_params, r, prep_qt, conv_qt, xe_pos=False):
    B, N, C = x.shape
    NS = posq.shape[1]
    (W1, b1), (W2, b2), (W3, b3) = mlp_params
    H1 = W1.shape[1]
    summary, f, o = _sa_prep(posq, pos, x, W1, b1, r, prep_qt)
    if xe_pos:
        table = jnp.pad(pos.reshape(B * N, 3), ((0, 0), (0, 13)))
        GW = 16
    else:
        table = f.reshape(B * N, H1)
        GW = H1
    G, vmask = _sa_sc_gather(
        posq[:, :, 0].reshape(-1), posq[:, :, 1].reshape(-1),
        posq[:, :, 2].reshape(-1),
        pos[:, :, 0].reshape(-1), pos[:, :, 1].reshape(-1),
        pos[:, :, 2].reshape(-1),
        summary.reshape(-1), table,
        B=B, N=N, NS=NS, NCH=N // 16, H1=GW, r2=r * r)
    wab = (W1[:C], W1[C:]) if xe_pos else None
    out = _sa_conv(G, o.reshape(B * NS, H1), vmask.reshape(B * NS, 64),
                   W2, b2, W3, b3, conv_qt, wab=wab)
    return out.reshape(B, NS, W3.shape[1])


def _td_body(x_ref, qc_ref, pt_ref, w_ref, b_ref, out_ref, *, k):
    x = x_ref[0]
    qc = qc_ref[0]
    pt = pt_ref[0]
    NS = qc.shape[0]
    N = x.shape[0]
    h = jnp.maximum(
        jax.lax.dot_general(x, w_ref[...], (((1,), (0,)), ((), ())),
                            preferred_element_type=jnp.float32) + b_ref[...], 0.0)
    d2 = ((qc[:, 0:1] - pt[0:1, :]) ** 2 + (qc[:, 1:2] - pt[1:2, :]) ** 2
          + (qc[:, 2:3] - pt[2:3, :]) ** 2)
    iota = jax.lax.broadcasted_iota(jnp.int32, (NS, N), 1)
    out = jnp.full((NS, h.shape[1]), -jnp.inf, dtype=jnp.float32)
    for _ in range(k):
        m = jnp.min(d2, axis=1, keepdims=True)
        j = jnp.min(jnp.where(d2 == m, iota, N), axis=1, keepdims=True)
        onehot = (iota == j).astype(jnp.float32)
        g = jax.lax.dot_general(onehot, h, (((1,), (0,)), ((), ())),
                                preferred_element_type=jnp.float32)
        out = jnp.maximum(out, g)
        d2 = jnp.where(iota == j, jnp.inf, d2)
    out_ref[0] = out


def _td_pallas(x, pos, posq, mlp_params, k=16):
    B, N, C = x.shape
    NS = posq.shape[1]
    (W, b) = mlp_params[0]
    H = W.shape[1]
    pos_t = pos.transpose(0, 2, 1)
    out = pl.pallas_call(
        functools.partial(_td_body, k=k),
        grid=(B,),
        in_specs=[
            pl.BlockSpec((1, N, C), lambda b_: (b_, 0, 0)),
            pl.BlockSpec((1, NS, 3), lambda b_: (b_, 0, 0)),
            pl.BlockSpec((1, 3, N), lambda b_: (b_, 0, 0)),
            pl.BlockSpec((C, H), lambda b_: (0, 0)),
            pl.BlockSpec((1, H), lambda b_: (0, 0)),
        ],
        out_specs=pl.BlockSpec((1, NS, H), lambda b_: (b_, 0, 0)),
        out_shape=jax.ShapeDtypeStruct((B, NS, H), jnp.float32),
    )(x, posq, pos_t, W, b.reshape(1, H))
    return out


def _cheb_body(x_ref, pc_ref, pt_ref, w0_ref, w1_ref, w2_ref, b_ref, eye_ref,
               out_ref, *, k):
    x = x_ref[0]
    pc = pc_ref[0]
    pt = pt_ref[0]
    N = x.shape[0]
    d2 = ((pc[:, 0:1] - pt[0:1, :]) ** 2 + (pc[:, 1:2] - pt[1:2, :]) ** 2
          + (pc[:, 2:3] - pt[2:3, :]) ** 2)
    iota = jax.lax.broadcasted_iota(jnp.int32, (N, N), 1)
    adj = jnp.zeros((N, N), dtype=jnp.float32)
    for _ in range(k):
        m = jnp.min(d2, axis=1, keepdims=True)
        j = jnp.min(jnp.where(d2 == m, iota, N), axis=1, keepdims=True)
        adj = adj + (iota == j).astype(jnp.float32)
        d2 = jnp.where(iota == j, jnp.inf, d2)
    eye = eye_ref[...]
    adj = adj * (1.0 - eye)
    deg = jnp.sum(adj, axis=1, keepdims=True)
    dinv = jnp.where(deg > 0.0, jax.lax.rsqrt(jnp.maximum(deg, 1e-12)), 0.0)
    dinv_row = jax.lax.dot_general(dinv, eye, (((0,), (0,)), ((), ())),
                                   preferred_element_type=jnp.float32)
    a = -dinv * dinv_row * adj

    def lhat(y):
        return jax.lax.dot_general(a, y, (((0,), (0,)), ((), ())),
                                   preferred_element_type=jnp.float32)

    tx1 = lhat(x)
    tx2 = 2.0 * lhat(tx1) - x

    def mm(u, wref):
        return jax.lax.dot_general(u, wref[...], (((1,), (0,)), ((), ())),
                                   preferred_element_type=jnp.float32)

    out_ref[0] = mm(x, w0_ref) + mm(tx1, w1_ref) + mm(tx2, w2_ref) + b_ref[...]


def _cheb_pallas(x, pos, params, k=16):
    B, N, C = x.shape
    Ws = params["Ws"]
    H = Ws[0].shape[1]
    pos_t = pos.transpose(0, 2, 1)
    eye = jnp.eye(N, dtype=jnp.float32)
    out = pl.pallas_call(
        functools.partial(_cheb_body, k=k),
        grid=(B,),
        in_specs=[
            pl.BlockSpec((1, N, C), lambda b_: (b_, 0, 0)),
            pl.BlockSpec((1, N, 3), lambda b_: (b_, 0, 0)),
            pl.BlockSpec((1, 3, N), lambda b_: (b_, 0, 0)),
            pl.BlockSpec((C, H), lambda b_: (0, 0)),
            pl.BlockSpec((C, H), lambda b_: (0, 0)),
            pl.BlockSpec((C, H), lambda b_: (0, 0)),
            pl.BlockSpec((1, H), lambda b_: (0, 0)),
            pl.BlockSpec((N, N), lambda b_: (0, 0)),
        ],
        out_specs=pl.BlockSpec((1, N, H), lambda b_: (b_, 0, 0)),
        out_shape=jax.ShapeDtypeStruct((B, N, H), jnp.float32),
    )(x, pos, pos_t, Ws[0], Ws[1], Ws[2], params["b"].reshape(1, H), eye)
    return out


def _head_body(x_ref, *refs):
    wrefs = refs[:-1]
    out_ref = refs[-1]
    h = x_ref[...]
    nw = len(wrefs) // 2

    def mm(u, wref):
        return jax.lax.dot_general(u, wref[...], (((1,), (0,)), ((), ())),
                                   preferred_element_type=jnp.float32)

    for li in range(3):
        h = mm(h, wrefs[2 * li]) + wrefs[2 * li + 1][...]
        if li < 2:
            h = jnp.maximum(h, 0.0)
    M = h.shape[0]
    B = M // 32
    g = jnp.mean(h.reshape(B, 32, h.shape[1]), axis=1)
    for li in range(3, nw):
        g = mm(g, wrefs[2 * li]) + wrefs[2 * li + 1][...]
        if li < nw - 1:
            g = jnp.maximum(g, 0.0)
    out_ref[...] = g


def _head_pallas(xp, sa3_params, head_params):
    B = xp.shape[0]
    X = xp.reshape(B * 32, xp.shape[2])
    wlist = []
    specs = [pl.BlockSpec(X.shape, lambda: (0, 0))]
    for (W, b) in list(sa3_params) + list(head_params):
        wlist += [W, b.reshape(1, W.shape[1])]
        specs += [pl.BlockSpec(W.shape, lambda: (0, 0)),
                  pl.BlockSpec((1, W.shape[1]), lambda: (0, 0))]
    H = wlist[-1].shape[1]
    return pl.pallas_call(
        _head_body,
        in_specs=specs,
        out_specs=pl.BlockSpec((B, H), lambda: (0, 0)),
        out_shape=jax.ShapeDtypeStruct((B, H), jnp.float32),
    )(X, *wlist)


def _mlp(params, x, plain_last=True):
    n = len(params)
    for i, (W, b) in enumerate(params):
        x = x @ W + b
        if (i < n - 1) or (not plain_last):
            x = jax.nn.relu(x)
    return x


def _fps(pos, n_sample):
    pos = jax.lax.stop_gradient(pos)
    N = pos.shape[0]
    idxs = jnp.zeros((n_sample,), dtype=jnp.int32)
    dists = jnp.full((N,), jnp.inf, dtype=jnp.float32)

    def body(i, carry):
        idxs, dists = carry
        last = pos[idxs[i - 1]]
        d = jnp.sum((pos - last) ** 2, axis=-1)
        dists = jnp.minimum(dists, d)
        nxt = jnp.argmax(dists).astype(jnp.int32)
        return idxs.at[i].set(nxt), dists

    idxs, _ = jax.lax.fori_loop(1, n_sample, body, (idxs, dists))
    return idxs


def _pairwise_sq(a, b):
    return jnp.sum((a[:, None, :] - b[None, :, :]) ** 2, axis=-1)


def _sa(x, pos, posq, mlp_params, r, max_nbr=64):
    d2 = _pairwise_sq(posq, pos)
    neg = jnp.where(d2 <= r * r, -d2, -jnp.inf)
    vals, nbr = jax.lax.top_k(neg, max_nbr)
    valid = vals > -jnp.inf
    msg = jnp.concatenate([x[nbr], pos[nbr] - posq[:, None, :]], axis=-1)
    h = _mlp(mlp_params, msg, plain_last=True)
    h = jnp.where(valid[:, :, None], h, -jnp.inf)
    out = jnp.max(h, axis=1)
    return out, posq


def _td(x, pos, posq, mlp_params, k=16):
    h = _mlp(mlp_params, x, plain_last=False)
    d2 = _pairwise_sq(posq, pos)
    _, nbr = jax.lax.top_k(-d2, k)
    out = jnp.max(h[nbr], axis=1)
    return out, posq


def _cheb(x, pos, params, k=16):
    n = x.shape[0]
    d2 = _pairwise_sq(pos, pos)
    _, nbr = jax.lax.top_k(-d2, k)
    src = jnp.repeat(jnp.arange(n), k)
    dst = nbr.reshape(-1)
    mask = (src != dst).astype(jnp.float32)
    deg = jax.ops.segment_sum(mask, src, num_segments=n)
    dinv = jnp.where(deg > 0, jax.lax.rsqrt(jnp.maximum(deg, 1e-12)), 0.0)
    w = -dinv[src] * dinv[dst] * mask

    def lhat(y):
        return jax.ops.segment_sum(w[:, None] * y[src], dst, num_segments=n)

    tx0 = x
    tx1 = lhat(tx0)
    tx2 = 2.0 * lhat(tx1) - tx0
    Ws = params["Ws"]
    return tx0 @ Ws[0] + tx1 @ Ws[1] + tx2 @ Ws[2] + params["b"]


def kernel(data, params):
    pos = data
    x = data
    q1 = _fps_pallas(pos, 2048)
    x = _sa_pallas(x, pos, q1, params["sa1"], 0.2, prep_qt=256, conv_qt=256, xe_pos=True)
    pos = q1
    q2 = _fps_pallas(pos, 512)
    x = _td_pallas(x, pos, q2, params["td1"])
    pos = q2
    x = _cheb_pallas(x, pos, params["cheb1"])
    q3 = _fps_pallas(pos, 128)
    x = _sa_pallas(x, pos, q3, params["sa2"], 0.4, prep_qt=128, conv_qt=128)
    pos = q3
    q4 = _fps_pallas(pos, 32)
    x = _td_pallas(x, pos, q4, params["td2"])
    pos = q4
    x = _cheb_pallas(x, pos, params["cheb2"])
    y = _head_pallas(jnp.concatenate([x, pos], axis=-1), params["sa3"], params["head"])
    return y
